# baseline (speedup 1.0000x reference)
import functools

import jax
import jax.numpy as jnp
from jax import lax
from jax.experimental import pallas as pl
from jax.experimental.pallas import tpu as pltpu
from jax.experimental.pallas import tpu_sc as plsc

_FIELD_DIM = 100000
_F = 26
_D = 32
_B = 16384
_N = _B * _F
_V = _FIELD_DIM * _F
_NC = 2
_NS = 16
_NW = _NC * _NS
_L = 16

_TW = 8192
_S = _TW // 4
_SLOG = _S.bit_length() - 1
_NBLK = (_V + _TW - 1) // _TW
_VP = _NBLK * _TW


def _repack_body(t_ref, o_ref):
    blk = t_ref[...]
    for q in range(4):
        o_ref[:, 32 * q:32 * (q + 1)] = blk[:, _S * q:_S * (q + 1)].T


def _repack(table_t):
    return pl.pallas_call(
        _repack_body,
        grid=(_NBLK,),
        in_specs=[pl.BlockSpec((_D, _TW), lambda j: (0, j))],
        out_specs=pl.BlockSpec((_TW // 4, 128), lambda j: (j, 0)),
        out_shape=jax.ShapeDtypeStruct((_VP * _D // 128, 128), jnp.float32),
    )(table_t)


_NU = _F * (_B // 128)
_UPW = _NU // _NW


def _sc_body(xt_hbm, tab_hbm, out_hbm,
             idx0, idx1, rows0, rows1, t0, t1,
             si0, si1, sg0, sg1, so0, so1):
    wid = lax.axis_index("s") * _NC + lax.axis_index("c")
    u0 = wid * _UPW
    ulast = u0 + _UPW - 1
    lanes = lax.iota(jnp.int32, _L)

    idx_b = (idx0, idx1)
    rows_b = (rows0, rows1)
    t_b = (t0, t1)
    si_b = (si0, si1)
    sg_b = (sg0, sg1)
    so_b = (so0, so1)

    def xoff(u):
        return pl.multiple_of(((u >> 7) << 14) + ((u & 127) << 7), 128)

    def start_idx(u, p):
        pltpu.make_async_copy(
            xt_hbm.at[pl.ds(xoff(u), 128)], idx_b[p], si_b[p]).start()

    def wait_idx(p):
        pltpu.make_async_copy(
            xt_hbm.at[pl.ds(0, 128)], idx_b[p], si_b[p]).wait()

    def remap(u, p):
        off = (u >> 7) * _FIELD_DIM
        ib = idx_b[p]
        for g in range(8):
            sl = pl.ds(_L * g, _L)
            v = ib[sl] + off
            ib[sl] = ((v & ~(_TW - 1)) + ((v & (_S - 1)) << 2)) + (
                (v & (_TW - 1)) >> _SLOG)

    def start_gather(p):
        pltpu.make_async_copy(tab_hbm.at[idx_b[p]], rows_b[p], sg_b[p]).start()

    def wait_gather(p):
        pltpu.make_async_copy(
            tab_hbm.at[pl.ds(0, 128)], rows_b[p], sg_b[p]).wait()

    def transpose(p):
        rb, tb = rows_b[p], t_b[p]

        def dstep(d, _):
            dsp = jnp.broadcast_to(d, (_L,))
            base = pl.multiple_of(d * 128, 128)
            for g in range(8):
                vals = plsc.load_gather(rb, [lanes + _L * g, dsp])
                tb[pl.ds(base + _L * g, _L)] = vals
            return 0

        lax.fori_loop(0, _D, dstep, 0)

    def start_outs(u, p):
        f = u >> 7
        q = u & 127
        for t in range(4):
            word0 = pl.multiple_of((((f * 4 + t) * 128) + q) * 1024, 1024)
            pltpu.make_async_copy(
                t_b[p].at[pl.ds(1024 * t, 1024)],
                out_hbm.at[pl.ds(word0, 1024)], so_b[p]).start()

    def wait_outs(p):
        for t in range(4):
            pltpu.make_async_copy(
                out_hbm.at[pl.ds(1024 * t, 1024)],
                t_b[p].at[pl.ds(1024 * t, 1024)], so_b[p]).wait()

    start_idx(u0, 0)
    start_idx(u0 + 1, 1)
    wait_idx(0)
    remap(u0, 0)
    start_gather(0)

    def half(u, p):
        @pl.when(u < ulast)
        def _():
            wait_idx(1 - p)
            remap(u + 1, 1 - p)
            start_gather(1 - p)

        wait_gather(p)

        @pl.when(u + 2 <= ulast)
        def _():
            start_idx(u + 2, p)

        @pl.when(u >= u0 + 2)
        def _():
            wait_outs(p)

        transpose(p)
        start_outs(u, p)

    def step(k, _):
        u = u0 + 2 * k
        half(u, 0)
        half(u + 1, 1)
        return 0

    lax.fori_loop(0, _UPW // 2, step, 0)
    wait_outs(0)
    wait_outs(1)


_sc_call = functools.partial(
    pl.kernel,
    out_type=jax.ShapeDtypeStruct((_N * _D,), jnp.float32),
    scratch_types=[
        pltpu.VMEM((128,), jnp.int32),
        pltpu.VMEM((128,), jnp.int32),
        pltpu.VMEM((128, _D), jnp.float32),
        pltpu.VMEM((128, _D), jnp.float32),
        pltpu.VMEM((4096,), jnp.float32),
        pltpu.VMEM((4096,), jnp.float32),
        pltpu.SemaphoreType.DMA,
        pltpu.SemaphoreType.DMA,
        pltpu.SemaphoreType.DMA,
        pltpu.SemaphoreType.DMA,
        pltpu.SemaphoreType.DMA,
        pltpu.SemaphoreType.DMA,
    ],
    mesh=plsc.VectorSubcoreMesh(core_axis_name="c", subcore_axis_name="s"),
    compiler_params=pltpu.CompilerParams(
        use_tc_tiling_on_sc=False, needs_layout_passes=False),
)(_sc_body)


def kernel(x, table):
    packed = _repack(table.T)
    tab_rm = packed.reshape(_VP * _D).reshape(_VP, _D)
    xt = x.T.reshape(_N)
    flat = _sc_call(xt, tab_rm)
    o5 = flat.reshape(_F, 4, 128, 8, 128)
    return o5.transpose(2, 4, 0, 1, 3).reshape(_B, _F, _D)
